# Initial kernel scaffold; baseline (speedup 1.0000x reference)
import jax
import jax.numpy as jnp
from jax.experimental import pallas as pl


def kernel(u, v, z, J, B, beta, gamma):
    raise NotImplementedError("write your pallas kernel here")


# speedup vs baseline: 6.8396x; 6.8396x over previous
import functools

import jax
import jax.numpy as jnp
import numpy as np
from jax import lax
from jax.experimental import pallas as pl
from jax.experimental.pallas import tpu as pltpu
from jax.experimental.pallas import tpu_sc as plsc

_L = 16

_PJ = np.array([[(16 * j + l) // 3 for l in range(_L)] for j in range(3)],
               dtype=np.int32)
_CJ = np.array([[(16 * j + l) % 3 for l in range(_L)] for j in range(3)],
               dtype=np.int32)


@functools.partial(jax.jit, static_argnames=("n", "w", "c_chunk"))
def _run(u1, v1, z1, jf, pats, pcj, *, n, w, c_chunk):
    NC, NS = 2, 16
    NW = NC * NS
    ppw = n // NW
    nchunks = ppw // c_chunk
    C = c_chunk

    mesh = plsc.VectorSubcoreMesh(core_axis_name="c", subcore_axis_name="s")

    @functools.partial(
        pl.kernel,
        mesh=mesh,
        out_type=jax.ShapeDtypeStruct((3 * n,), jnp.float32),
        scratch_types=[
            pltpu.VMEM((C,), jnp.int32),
            pltpu.VMEM((C,), jnp.int32),
            pltpu.VMEM((C,), jnp.float32),
            pltpu.VMEM((3 * C,), jnp.int32),
            pltpu.VMEM((3 * C,), jnp.float32),
            pltpu.VMEM((3 * C,), jnp.float32),
            pltpu.VMEM((3, 48), jnp.float32),
            pltpu.VMEM((6, 16), jnp.int32),
            pltpu.SemaphoreType.DMA,
        ],
    )
    def kern(u_hbm, v_hbm, z_hbm, j_hbm, p_hbm, pcj_hbm, out_hbm,
             ubuf, vbuf, zbuf, idxbuf, z3buf, gbuf, patbuf, pcjbuf, sem):
        wid = lax.axis_index("s") * NC + lax.axis_index("c")
        base = wid * ppw
        pltpu.sync_copy(p_hbm, patbuf)
        pltpu.sync_copy(pcj_hbm, pcjbuf)

        dnums = lax.GatherDimensionNumbers(
            offset_dims=(), collapsed_slice_dims=(0,), start_index_map=(0,))

        def take16(vec, lanes):
            return lax.gather(vec, lanes[:, None], dnums, slice_sizes=(1,),
                              mode=lax.GatherScatterMode.PROMISE_IN_BOUNDS)

        def pass_idx(m, carry):
            uu16 = ubuf[pl.ds(m * 16, 16)]
            vv16 = vbuf[pl.ds(m * 16, 16)]
            zz16 = zbuf[pl.ds(m * 16, 16)]
            f16 = (vv16 * w + uu16) * 3
            for j in range(3):
                pjv = pcjbuf[j, :]
                idx = take16(f16, pjv) + pcjbuf[3 + j, :]
                zz = take16(zz16, pjv)
                pos0 = m * 48 + 16 * j
                idxbuf[pl.ds(pos0, 16)] = idx
                z3buf[pl.ds(pos0, 16)] = zz
            return carry

        def pass_model(m, carry):
            for j in range(3):
                pos0 = m * 48 + 16 * j
                g = gbuf[pl.ds(pos0, 16)]
                zz = z3buf[pl.ds(pos0, 16)]
                nb = patbuf[0, pl.ds(16 * j, 16)]
                ng = patbuf[1, pl.ds(16 * j, 16)]
                bp = patbuf[2, pl.ds(16 * j, 16)]
                e1 = jnp.exp(zz * nb)
                e2 = jnp.exp(zz * ng)
                gbuf[pl.ds(pos0, 16)] = g * e1 + bp * (1.0 - e2)
            return carry

        def chunk(k, carry):
            p0 = base + k * C
            pltpu.sync_copy(u_hbm.at[pl.ds(p0, C)], ubuf)
            pltpu.sync_copy(v_hbm.at[pl.ds(p0, C)], vbuf)
            pltpu.sync_copy(z_hbm.at[pl.ds(p0, C)], zbuf)
            lax.fori_loop(0, C // 16, pass_idx, 0)
            pltpu.async_copy(j_hbm.at[idxbuf], gbuf, sem).wait()
            lax.fori_loop(0, C // 16, pass_model, 0)
            pltpu.sync_copy(gbuf, out_hbm.at[pl.ds(3 * p0, 3 * C)])
            return carry

        lax.fori_loop(0, nchunks, chunk, 0)

    return kern(u1, v1, z1, jf, pats, pcj)


def kernel(u, v, z, J, B, beta, gamma):
    n = u.shape[0]
    h, w, _ = J.shape
    u1 = u.astype(jnp.int32)
    v1 = v.astype(jnp.int32)
    jf = J.reshape(h * w * 3)
    pats = jnp.stack([
        jnp.tile(-beta.astype(jnp.float32), 16),
        jnp.tile(-gamma.astype(jnp.float32), 16),
        jnp.tile(B.astype(jnp.float32), 16),
    ])
    pcj = jnp.asarray(np.concatenate([_PJ, _CJ], axis=0))
    out = _run(u1, v1, z, jf, pats, pcj, n=n, w=w, c_chunk=8192)
    return out.reshape(n, 3)


# speedup vs baseline: 69.5139x; 10.1634x over previous
import functools

import jax
import jax.numpy as jnp
from jax import lax
from jax.experimental import pallas as pl
from jax.experimental.pallas import tpu as pltpu
from jax.experimental.pallas import tpu_sc as plsc


@functools.partial(jax.jit, static_argnames=("n", "h", "w", "c_chunk"))
def _run(u1, v1, z1, jraw, pats, *, n, h, w, c_chunk):
    NC, NS = 2, 16
    NW = NC * NS
    ppw = n // NW
    nchunks = ppw // c_chunk
    C = c_chunk
    ntx = (w + 127) // 128
    planewords = ((h + 7) // 8) * ntx * 1024

    mesh = plsc.VectorSubcoreMesh(core_axis_name="c", subcore_axis_name="s")

    @functools.partial(
        pl.kernel,
        mesh=mesh,
        out_type=jax.ShapeDtypeStruct((n // 128, 4, 128), jnp.float32),
        scratch_types=[
            pltpu.VMEM((C,), jnp.int32),
            pltpu.VMEM((C,), jnp.int32),
            pltpu.VMEM((C,), jnp.float32),
            pltpu.VMEM((3 * C,), jnp.int32),
            pltpu.VMEM((3 * C,), jnp.float32),
            pltpu.VMEM((C // 128, 4, 128), jnp.float32),
            pltpu.VMEM((9, 16), jnp.float32),
            pltpu.SemaphoreType.DMA,
        ],
    )
    def kern(u_hbm, v_hbm, z_hbm, j_hbm, p_hbm, out_hbm,
             ubuf, vbuf, zbuf, idxbuf, gbuf, obuf, patbuf, sem):
        wid = lax.axis_index("s") * NC + lax.axis_index("c")
        base = wid * ppw
        pltpu.sync_copy(p_hbm, patbuf)

        def pass_idx(t, carry):
            g = lax.shift_right_logical(t, 3)
            b16 = lax.bitwise_and(t, 7) * 16
            uu = ubuf[pl.ds(t * 16, 16)]
            vv = vbuf[pl.ds(t * 16, 16)]
            ty = lax.shift_right_logical(vv, 3)
            ry = lax.bitwise_and(vv, 7)
            tx = lax.shift_right_logical(uu, 7)
            rx = lax.bitwise_and(uu, 127)
            base_idx = (ty * ntx + tx) * 1024 + ry * 128 + rx
            dst0 = g * 384 + b16
            idxbuf[pl.ds(dst0, 16)] = base_idx
            idxbuf[pl.ds(dst0 + 128, 16)] = base_idx + planewords
            idxbuf[pl.ds(dst0 + 256, 16)] = base_idx + 2 * planewords
            return carry

        def pass_model(t, carry):
            g = lax.shift_right_logical(t, 3)
            b16 = lax.bitwise_and(t, 7) * 16
            zz = zbuf[pl.ds(t * 16, 16)]
            src0 = g * 384 + b16
            for c in range(3):
                gval = gbuf[pl.ds(src0 + c * 128, 16)]
                nb = patbuf[c, :]
                ng = patbuf[3 + c, :]
                bp = patbuf[6 + c, :]
                e1 = jnp.exp(zz * nb)
                e2 = jnp.exp(zz * ng)
                obuf[g, c, pl.ds(b16, 16)] = gval * e1 + bp * (1.0 - e2)
            return carry

        def chunk(k, carry):
            p0 = base + k * C
            pltpu.sync_copy(u_hbm.at[pl.ds(p0, C)], ubuf)
            pltpu.sync_copy(v_hbm.at[pl.ds(p0, C)], vbuf)
            pltpu.sync_copy(z_hbm.at[pl.ds(p0, C)], zbuf)
            lax.fori_loop(0, C // 16, pass_idx, 0)
            pltpu.async_copy(j_hbm.at[idxbuf], gbuf, sem).wait()
            lax.fori_loop(0, C // 16, pass_model, 0)
            pltpu.sync_copy(obuf, out_hbm.at[pl.ds(p0 // 128, C // 128)])
            return carry

        lax.fori_loop(0, nchunks, chunk, 0)

    return kern(u1, v1, z1, jraw, pats)


def kernel(u, v, z, J, B, beta, gamma):
    n = u.shape[0]
    h, w, _ = J.shape
    th, tw = (h + 7) // 8, (w + 127) // 128
    jraw = (J.transpose(2, 0, 1)
             .reshape(3, th, 8, tw, 128)
             .transpose(0, 1, 3, 2, 4)
             .reshape(3 * th * tw * 8 * 128))
    pats = jnp.repeat(
        jnp.concatenate([-beta, -gamma, B]).astype(jnp.float32)[:, None],
        16, axis=1)
    out4 = _run(u.astype(jnp.int32), v.astype(jnp.int32), z, jraw, pats,
                n=n, h=h, w=w, c_chunk=8192)
    return out4[:, :3, :].transpose(0, 2, 1).reshape(n, 3)


# speedup vs baseline: 118.6728x; 1.7072x over previous
import functools

import jax
import jax.numpy as jnp
from jax import lax
from jax.experimental import pallas as pl
from jax.experimental.pallas import tpu as pltpu
from jax.experimental.pallas import tpu_sc as plsc


@functools.partial(jax.jit, static_argnames=("n", "h", "w", "c_chunk"))
def _run(u1, v1, z1, jraw, pats, *, n, h, w, c_chunk):
    NC, NS = 2, 16
    NW = NC * NS
    ppw = n // NW
    nchunks = ppw // c_chunk
    C = c_chunk
    ntx = (w + 127) // 128
    planewords = ((h + 7) // 8) * ntx * 1024

    mesh = plsc.VectorSubcoreMesh(core_axis_name="c", subcore_axis_name="s")

    vmem_set = [
        pltpu.VMEM((C,), jnp.int32),
        pltpu.VMEM((C,), jnp.int32),
        pltpu.VMEM((C,), jnp.float32),
        pltpu.VMEM((3 * C,), jnp.int32),
        pltpu.VMEM((3 * C,), jnp.float32),
        pltpu.VMEM((C // 128, 4, 128), jnp.float32),
    ]

    @functools.partial(
        pl.kernel,
        mesh=mesh,
        out_type=jax.ShapeDtypeStruct((n // 128, 4, 128), jnp.float32),
        scratch_types=vmem_set + vmem_set + [
            pltpu.VMEM((9, 16), jnp.float32),
            pltpu.SemaphoreType.DMA,
            pltpu.SemaphoreType.DMA,
            pltpu.SemaphoreType.DMA,
            pltpu.SemaphoreType.DMA,
        ],
    )
    def kern(u_hbm, v_hbm, z_hbm, j_hbm, p_hbm, out_hbm,
             u0, v0, z0, i0, g0, o0, u1b, v1b, z1b, i1, g1, o1,
             patbuf, sg0, sg1, si0, si1):
        wid = lax.axis_index("s") * NC + lax.axis_index("c")
        base = wid * ppw
        pltpu.sync_copy(p_hbm, patbuf)

        sets = ((u0, v0, z0, i0, g0, o0, sg0, si0),
                (u1b, v1b, z1b, i1, g1, o1, sg1, si1))

        def make_pass_idx(ub, vb, ib):
            def pass_idx(t, carry):
                g = lax.shift_right_logical(t, 3)
                b16 = lax.bitwise_and(t, 7) * 16
                uu = ub[pl.ds(t * 16, 16)]
                vv = vb[pl.ds(t * 16, 16)]
                ty = lax.shift_right_logical(vv, 3)
                ry = lax.bitwise_and(vv, 7)
                tx = lax.shift_right_logical(uu, 7)
                rx = lax.bitwise_and(uu, 127)
                base_idx = (ty * ntx + tx) * 1024 + ry * 128 + rx
                dst0 = g * 384 + b16
                ib[pl.ds(dst0, 16)] = base_idx
                ib[pl.ds(dst0 + 128, 16)] = base_idx + planewords
                ib[pl.ds(dst0 + 256, 16)] = base_idx + 2 * planewords
                return carry
            return pass_idx

        def make_pass_model(zb, gb, ob):
            def pass_model(t, carry):
                g = lax.shift_right_logical(t, 3)
                b16 = lax.bitwise_and(t, 7) * 16
                zz = zb[pl.ds(t * 16, 16)]
                src0 = g * 384 + b16
                for c in range(3):
                    gval = gb[pl.ds(src0 + c * 128, 16)]
                    nb = patbuf[c, :]
                    ng = patbuf[3 + c, :]
                    bp = patbuf[6 + c, :]
                    e1 = jnp.exp(zz * nb)
                    e2 = jnp.exp(zz * ng)
                    ob[g, c, pl.ds(b16, 16)] = gval * e1 + bp * (1.0 - e2)
                return carry
            return pass_model

        def load_and_launch(k, s, sync_inputs):
            ub, vb, zb, ib, gb, ob, sg, si = s
            p0 = base + k * C
            if sync_inputs:
                pltpu.sync_copy(u_hbm.at[pl.ds(p0, C)], ub)
                pltpu.sync_copy(v_hbm.at[pl.ds(p0, C)], vb)
                pltpu.sync_copy(z_hbm.at[pl.ds(p0, C)], zb)
            else:
                hu = pltpu.async_copy(u_hbm.at[pl.ds(p0, C)], ub, si)
                hv = pltpu.async_copy(v_hbm.at[pl.ds(p0, C)], vb, si)
                hz = pltpu.async_copy(z_hbm.at[pl.ds(p0, C)], zb, si)
                hu.wait()
                hv.wait()
                hz.wait()
            lax.fori_loop(0, C // 16, make_pass_idx(ub, vb, ib), 0)
            pltpu.async_copy(j_hbm.at[ib], gb, sg)

        def finish(k, s):
            ub, vb, zb, ib, gb, ob, sg, si = s
            pltpu.make_async_copy(j_hbm.at[ib], gb, sg).wait()
            lax.fori_loop(0, C // 16, make_pass_model(zb, gb, ob), 0)
            p0 = base + k * C
            pltpu.sync_copy(ob, out_hbm.at[pl.ds(p0 // 128, C // 128)])

        load_and_launch(0, sets[0], True)
        load_and_launch(1, sets[1], True)

        def body(p, carry):
            for par in range(2):
                k = 2 * p + par
                finish(k, sets[par])
                load_and_launch(k + 2, sets[par], False)
            return carry

        lax.fori_loop(0, nchunks // 2 - 1, body, 0)
        finish(nchunks - 2, sets[0])
        finish(nchunks - 1, sets[1])

    return kern(u1, v1, z1, jraw, pats)


def kernel(u, v, z, J, B, beta, gamma):
    n = u.shape[0]
    h, w, _ = J.shape
    th, tw = (h + 7) // 8, (w + 127) // 128
    jraw = (J.transpose(2, 0, 1)
             .reshape(3, th, 8, tw, 128)
             .transpose(0, 1, 3, 2, 4)
             .reshape(3 * th * tw * 8 * 128))
    pats = jnp.repeat(
        jnp.concatenate([-beta, -gamma, B]).astype(jnp.float32)[:, None],
        16, axis=1)
    out4 = _run(u.astype(jnp.int32), v.astype(jnp.int32), z, jraw, pats,
                n=n, h=h, w=w, c_chunk=4096)
    return out4[:, :3, :].transpose(0, 2, 1).reshape(n, 3)


# speedup vs baseline: 130.1405x; 1.0966x over previous
import functools

import jax
import jax.numpy as jnp
from jax import lax
from jax.experimental import pallas as pl
from jax.experimental.pallas import tpu as pltpu
from jax.experimental.pallas import tpu_sc as plsc


@functools.partial(jax.jit, static_argnames=("n", "h", "w", "c_chunk"))
def _run(u1, v1, z1, jraw, pats, *, n, h, w, c_chunk):
    NC, NS = 2, 16
    NW = NC * NS
    ppw = n // NW
    nchunks = ppw // c_chunk
    C = c_chunk
    ntx = (w + 127) // 128
    planewords = ((h + 7) // 8) * ntx * 1024

    mesh = plsc.VectorSubcoreMesh(core_axis_name="c", subcore_axis_name="s")

    per_par = [
        pltpu.VMEM((C,), jnp.int32),
        pltpu.VMEM((C,), jnp.int32),
        pltpu.VMEM((3 * C,), jnp.int32),
        pltpu.VMEM((3 * C,), jnp.float32),
        pltpu.VMEM((C // 128, 4, 128), jnp.float32),
    ]
    zbufs = [pltpu.VMEM((C,), jnp.float32)] * 4

    @functools.partial(
        pl.kernel,
        mesh=mesh,
        out_type=jax.ShapeDtypeStruct((n // 128, 4, 128), jnp.float32),
        scratch_types=per_par + per_par + zbufs + [
            pltpu.VMEM((9, 16), jnp.float32),
            pltpu.SemaphoreType.DMA,
            pltpu.SemaphoreType.DMA,
            pltpu.SemaphoreType.DMA,
            pltpu.SemaphoreType.DMA,
        ],
    )
    def kern(u_hbm, v_hbm, z_hbm, j_hbm, p_hbm, out_hbm,
             ua, va, ia, ga, oa, ub, vb, ib, gb, ob,
             zq0, zq1, zq2, zq3,
             patbuf, sg0, sg1, si0, si1):
        wid = lax.axis_index("s") * NC + lax.axis_index("c")
        base = wid * ppw
        pltpu.sync_copy(p_hbm, patbuf)

        par_sets = ((ua, va, ia, ga, oa, sg0, si0),
                    (ub, vb, ib, gb, ob, sg1, si1))
        zsets = (zq0, zq1, zq2, zq3)

        def load_pats():
            return tuple(patbuf[r, :] for r in range(9))

        def idx_step(t, uref, vref, iref):
            g = lax.shift_right_logical(t, 3)
            b16 = lax.bitwise_and(t, 7) * 16
            uu = uref[pl.ds(t * 16, 16)]
            vv = vref[pl.ds(t * 16, 16)]
            ty = lax.shift_right_logical(vv, 3)
            ry = lax.bitwise_and(vv, 7)
            tx = lax.shift_right_logical(uu, 7)
            rx = lax.bitwise_and(uu, 127)
            base_idx = (ty * ntx + tx) * 1024 + ry * 128 + rx
            dst0 = g * 384 + b16
            iref[pl.ds(dst0, 16)] = base_idx
            iref[pl.ds(dst0 + 128, 16)] = base_idx + planewords
            iref[pl.ds(dst0 + 256, 16)] = base_idx + 2 * planewords

        def model_step(t, zref, gref, oref, pv):
            g = lax.shift_right_logical(t, 3)
            b16 = lax.bitwise_and(t, 7) * 16
            zz = zref[pl.ds(t * 16, 16)]
            src0 = g * 384 + b16
            for c in range(3):
                gval = gref[pl.ds(src0 + c * 128, 16)]
                e1 = jnp.exp(zz * pv[c])
                e2 = jnp.exp(zz * pv[3 + c])
                oref[g, c, pl.ds(b16, 16)] = gval * e1 + pv[6 + c] * (1.0 - e2)

        def launch(k, sync_inputs):
            uref, vref, iref, gref, _, sg, si = par_sets[k & 1]
            zref = zsets[k & 3]
            p0 = base + k * C
            if sync_inputs:
                pltpu.sync_copy(u_hbm.at[pl.ds(p0, C)], uref)
                pltpu.sync_copy(v_hbm.at[pl.ds(p0, C)], vref)
                pltpu.sync_copy(z_hbm.at[pl.ds(p0, C)], zref)

            def idx_only(t, carry):
                idx_step(t, uref, vref, iref)
                return carry

            lax.fori_loop(0, C // 16, idx_only, 0)
            pltpu.async_copy(j_hbm.at[iref], gref, sg)

        def full_body(k, j):
            uref, vref, iref, gref, oref, sg, si = par_sets[j & 1]
            zref = zsets[j & 3]
            z2ref = zsets[(j + 2) & 3]
            p0 = base + k * C
            p2 = p0 + 2 * C
            hu = pltpu.async_copy(u_hbm.at[pl.ds(p2, C)], uref, si)
            hv = pltpu.async_copy(v_hbm.at[pl.ds(p2, C)], vref, si)
            hz = pltpu.async_copy(z_hbm.at[pl.ds(p2, C)], z2ref, si)
            pltpu.make_async_copy(j_hbm.at[iref], gref, sg).wait()
            hu.wait()
            hv.wait()
            hz.wait()

            def fused(t, pv):
                model_step(t, zref, gref, oref, pv)
                idx_step(t, uref, vref, iref)
                return pv

            lax.fori_loop(0, C // 16, fused, load_pats())
            pltpu.async_copy(j_hbm.at[iref], gref, sg)
            pltpu.sync_copy(oref, out_hbm.at[pl.ds(p0 // 128, C // 128)])

        def tail_body(k):
            _, _, iref, gref, oref, sg, _ = par_sets[k & 1]
            zref = zsets[k & 3]
            pltpu.make_async_copy(j_hbm.at[iref], gref, sg).wait()

            def model_only(t, pv):
                model_step(t, zref, gref, oref, pv)
                return pv

            lax.fori_loop(0, C // 16, model_only, load_pats())
            p0 = base + k * C
            pltpu.sync_copy(oref, out_hbm.at[pl.ds(p0 // 128, C // 128)])

        launch(0, True)
        launch(1, True)

        def body(p2, carry):
            for j in range(4):
                full_body(4 * p2 + j, j)
            return carry

        lax.fori_loop(0, nchunks // 4 - 1, body, 0)
        full_body(nchunks - 4, nchunks - 4)
        full_body(nchunks - 3, nchunks - 3)
        tail_body(nchunks - 2)
        tail_body(nchunks - 1)

    return kern(u1, v1, z1, jraw, pats)


def kernel(u, v, z, J, B, beta, gamma):
    n = u.shape[0]
    h, w, _ = J.shape
    th, tw = (h + 7) // 8, (w + 127) // 128
    jraw = (J.transpose(2, 0, 1)
             .reshape(3, th, 8, tw, 128)
             .transpose(0, 1, 3, 2, 4)
             .reshape(3 * th * tw * 8 * 128))
    pats = jnp.repeat(
        jnp.concatenate([-beta, -gamma, B]).astype(jnp.float32)[:, None],
        16, axis=1)
    out4 = _run(u.astype(jnp.int32), v.astype(jnp.int32), z, jraw, pats,
                n=n, h=h, w=w, c_chunk=4096)
    return out4[:, :3, :].transpose(0, 2, 1).reshape(n, 3)
